# baseline (speedup 1.0000x reference)
import jax
import jax.numpy as jnp
from jax.experimental import pallas as pl
from jax.experimental.pallas import tpu as pltpu

_B_BLOCK = 1024
_NSPLIT = 2
_K = 1024
_DIM = 256
_NCB = 4


def _round(r, hi, cbcat, colenc):
    sims = jax.lax.dot_general(
        r.astype(jnp.bfloat16), hi, (((1,), (1,)), ((), ())),
        preferred_element_type=jnp.float32)
    m = jnp.max(sims, axis=1, keepdims=True)
    oh = (sims == m).astype(jnp.bfloat16)
    oh3 = jnp.concatenate([oh, oh, oh], axis=1)
    sel = jax.lax.dot_general(
        oh3, cbcat, (((1,), (0,)), ((), ())),
        preferred_element_type=jnp.float32)
    idxe = jax.lax.dot_general(
        oh3, colenc, (((1,), (0,)), ((), ())),
        preferred_element_type=jnp.float32)
    idx = (idxe[:, 0] + idxe[:, 1]).astype(jnp.int32)
    return r - sel, idx


def _rvq_body(q_ref, hi_ref, cbcat_ref, colenc_ref, idx_ref, emb_ref):
    sub = _B_BLOCK // _NSPLIT
    qs = [q_ref[pl.ds(s * sub, sub), :] for s in range(_NSPLIT)]
    rs = list(qs)
    colenc = colenc_ref[...]
    for i in range(_NCB):
        hi = hi_ref[i]
        cbcat = cbcat_ref[i]
        for s in range(_NSPLIT):
            rs[s], idx = _round(rs[s], hi, cbcat, colenc)
            idx_ref[i, pl.ds(s * sub, sub)] = idx
    for s in range(_NSPLIT):
        emb_ref[pl.ds(s * sub, sub), :] = qs[s] - rs[s]


@jax.jit
def kernel(query, cb0, cb1, cb2, cb3):
    B = query.shape[0]
    cbs = jnp.stack([cb0, cb1, cb2, cb3], axis=0)
    hi = cbs.astype(jnp.bfloat16)
    m16 = jnp.int32(-65536)
    t1f = jax.lax.bitcast_convert_type(
        jax.lax.bitcast_convert_type(cbs, jnp.int32) & m16, jnp.float32)
    r1 = cbs - t1f
    t2f = jax.lax.bitcast_convert_type(
        jax.lax.bitcast_convert_type(r1, jnp.int32) & m16, jnp.float32)
    r2 = r1 - t2f
    cbcat = jnp.concatenate(
        [t1f.astype(jnp.bfloat16), t2f.astype(jnp.bfloat16),
         r2.astype(jnp.bfloat16)], axis=1)
    col = jnp.arange(_K, dtype=jnp.int32)
    colenc = jnp.zeros((3 * _K, 128), jnp.float32)
    colenc = colenc.at[:_K, 0].set(((col >> 8) * 256).astype(jnp.float32))
    colenc = colenc.at[:_K, 1].set((col & 255).astype(jnp.float32))
    colenc = colenc.astype(jnp.bfloat16)
    grid = (B // _B_BLOCK,)
    idx, emb = pl.pallas_call(
        _rvq_body,
        grid=grid,
        in_specs=[
            pl.BlockSpec((_B_BLOCK, _DIM), lambda i: (i, 0)),
            pl.BlockSpec((_NCB, _K, _DIM), lambda i: (0, 0, 0)),
            pl.BlockSpec((_NCB, 3 * _K, _DIM), lambda i: (0, 0, 0)),
            pl.BlockSpec((3 * _K, 128), lambda i: (0, 0)),
        ],
        out_specs=[
            pl.BlockSpec((_NCB, _B_BLOCK), lambda i: (0, i)),
            pl.BlockSpec((_B_BLOCK, _DIM), lambda i: (i, 0)),
        ],
        out_shape=[
            jax.ShapeDtypeStruct((_NCB, B), jnp.int32),
            jax.ShapeDtypeStruct((B, _DIM), jnp.float32),
        ],
        compiler_params=pltpu.CompilerParams(
            dimension_semantics=("arbitrary",),
        ),
    )(query, hi, cbcat, colenc)
    return idx, emb


# speedup vs baseline: 1.3573x; 1.3573x over previous
import jax
import jax.numpy as jnp
from jax.experimental import pallas as pl
from jax.experimental.pallas import tpu as pltpu

_B_BLOCK = 1024
_NSPLIT = 4
_K = 1024
_DIM = 256
_NCB = 4


def _round(r, col, hi, cbcat):
    sims = jax.lax.dot_general(
        r.astype(jnp.bfloat16), hi, (((1,), (1,)), ((), ())),
        preferred_element_type=jnp.float32)
    m = jnp.max(sims, axis=1, keepdims=True)
    mask = sims == m
    idx = jnp.min(jnp.where(mask, col, _K), axis=1).astype(jnp.int32)
    oh = mask.astype(jnp.bfloat16)
    oh3 = jnp.concatenate([oh, oh, oh], axis=1)
    sel = jax.lax.dot_general(
        oh3, cbcat, (((1,), (0,)), ((), ())),
        preferred_element_type=jnp.float32)
    return r - sel, idx


def _rvq_body(q_ref, hi_ref, cbcat_ref, idx_ref, emb_ref):
    sub = _B_BLOCK // _NSPLIT
    qs = [q_ref[pl.ds(s * sub, sub), :] for s in range(_NSPLIT)]
    rs = list(qs)
    col = jax.lax.broadcasted_iota(jnp.int32, (sub, _K), 1)
    for i in range(_NCB):
        hi = hi_ref[i]
        cbcat = cbcat_ref[i]
        for s in range(_NSPLIT):
            rs[s], idx = _round(rs[s], col, hi, cbcat)
            idx_ref[i, pl.ds(s * sub, sub)] = idx
    for s in range(_NSPLIT):
        emb_ref[pl.ds(s * sub, sub), :] = qs[s] - rs[s]


@jax.jit
def kernel(query, cb0, cb1, cb2, cb3):
    B = query.shape[0]
    cbs = jnp.stack([cb0, cb1, cb2, cb3], axis=0)
    hi = cbs.astype(jnp.bfloat16)
    m16 = jnp.int32(-65536)
    t1f = jax.lax.bitcast_convert_type(
        jax.lax.bitcast_convert_type(cbs, jnp.int32) & m16, jnp.float32)
    r1 = cbs - t1f
    t2f = jax.lax.bitcast_convert_type(
        jax.lax.bitcast_convert_type(r1, jnp.int32) & m16, jnp.float32)
    r2 = r1 - t2f
    cbcat = jnp.concatenate(
        [t1f.astype(jnp.bfloat16), t2f.astype(jnp.bfloat16),
         r2.astype(jnp.bfloat16)], axis=1)
    grid = (B // _B_BLOCK,)
    idx, emb = pl.pallas_call(
        _rvq_body,
        grid=grid,
        in_specs=[
            pl.BlockSpec((_B_BLOCK, _DIM), lambda i: (i, 0)),
            pl.BlockSpec((_NCB, _K, _DIM), lambda i: (0, 0, 0)),
            pl.BlockSpec((_NCB, 3 * _K, _DIM), lambda i: (0, 0, 0)),
        ],
        out_specs=[
            pl.BlockSpec((_NCB, _B_BLOCK), lambda i: (0, i)),
            pl.BlockSpec((_B_BLOCK, _DIM), lambda i: (i, 0)),
        ],
        out_shape=[
            jax.ShapeDtypeStruct((_NCB, B), jnp.int32),
            jax.ShapeDtypeStruct((B, _DIM), jnp.float32),
        ],
        compiler_params=pltpu.CompilerParams(
            dimension_semantics=("arbitrary",),
        ),
    )(query, hi, cbcat)
    return idx, emb


# speedup vs baseline: 1.6097x; 1.1860x over previous
import jax
import jax.numpy as jnp
from jax.experimental import pallas as pl
from jax.experimental.pallas import tpu as pltpu

_B_BLOCK = 2048
_NSPLIT = 2
_K = 1024
_DIM = 256
_NCB = 4


def _round(r, col, hi, cbcat):
    sims = jax.lax.dot_general(
        r.astype(jnp.bfloat16), hi, (((1,), (1,)), ((), ())),
        preferred_element_type=jnp.float32)
    m = jnp.max(sims, axis=1, keepdims=True)
    mask = sims == m
    idx = jnp.min(jnp.where(mask, col, _K), axis=1).astype(jnp.int32)
    oh = mask.astype(jnp.bfloat16)
    oh3 = jnp.concatenate([oh, oh, oh], axis=1)
    sel = jax.lax.dot_general(
        oh3, cbcat, (((1,), (0,)), ((), ())),
        preferred_element_type=jnp.float32)
    return r - sel, idx


def _rvq_body(q_ref, hi_ref, cbcat_ref, idx_ref, emb_ref):
    sub = _B_BLOCK // _NSPLIT
    qs = [q_ref[pl.ds(s * sub, sub), :] for s in range(_NSPLIT)]
    rs = list(qs)
    col = jax.lax.broadcasted_iota(jnp.int32, (sub, _K), 1)
    for i in range(_NCB):
        hi = hi_ref[i]
        cbcat = cbcat_ref[i]
        for s in range(_NSPLIT):
            rs[s], idx = _round(rs[s], col, hi, cbcat)
            idx_ref[i, pl.ds(s * sub, sub)] = idx
    for s in range(_NSPLIT):
        emb_ref[pl.ds(s * sub, sub), :] = qs[s] - rs[s]


@jax.jit
def kernel(query, cb0, cb1, cb2, cb3):
    B = query.shape[0]
    cbs = jnp.stack([cb0, cb1, cb2, cb3], axis=0)
    hi = cbs.astype(jnp.bfloat16)
    m16 = jnp.int32(-65536)
    t1f = jax.lax.bitcast_convert_type(
        jax.lax.bitcast_convert_type(cbs, jnp.int32) & m16, jnp.float32)
    r1 = cbs - t1f
    t2f = jax.lax.bitcast_convert_type(
        jax.lax.bitcast_convert_type(r1, jnp.int32) & m16, jnp.float32)
    r2 = r1 - t2f
    cbcat = jnp.concatenate(
        [t1f.astype(jnp.bfloat16), t2f.astype(jnp.bfloat16),
         r2.astype(jnp.bfloat16)], axis=1)
    grid = (B // _B_BLOCK,)
    idx, emb = pl.pallas_call(
        _rvq_body,
        grid=grid,
        in_specs=[
            pl.BlockSpec((_B_BLOCK, _DIM), lambda i: (i, 0)),
            pl.BlockSpec((_NCB, _K, _DIM), lambda i: (0, 0, 0)),
            pl.BlockSpec((_NCB, 3 * _K, _DIM), lambda i: (0, 0, 0)),
        ],
        out_specs=[
            pl.BlockSpec((_NCB, _B_BLOCK), lambda i: (0, i)),
            pl.BlockSpec((_B_BLOCK, _DIM), lambda i: (i, 0)),
        ],
        out_shape=[
            jax.ShapeDtypeStruct((_NCB, B), jnp.int32),
            jax.ShapeDtypeStruct((B, _DIM), jnp.float32),
        ],
        compiler_params=pltpu.CompilerParams(
            dimension_semantics=("arbitrary",),
        ),
    )(query, hi, cbcat)
    return idx, emb
